# Initial kernel scaffold; baseline (speedup 1.0000x reference)
import jax
import jax.numpy as jnp
from jax.experimental import pallas as pl


def kernel(x, embeddings, W1, b1, W2, b2):
    raise NotImplementedError("write your pallas kernel here")


# speedup vs baseline: 8.3380x; 8.3380x over previous
import functools

import jax
import jax.numpy as jnp
from jax import lax
from jax.experimental import pallas as pl
from jax.experimental.pallas import tpu as pltpu
from jax.experimental.pallas import tpu_sc as plsc

B = 16384
H = 200
E = 32
HID = 128
OUT = 2

NC, NS = 2, 16
NW = NC * NS
RPG = 128
G = B // RPG
GPW = G // NW
IPG = RPG * H

_mesh = plsc.VectorSubcoreMesh(
    core_axis_name="c", subcore_axis_name="s", num_cores=NC, num_subcores=NS
)


@functools.partial(
    pl.kernel,
    out_type=jax.ShapeDtypeStruct((B, E), jnp.float32),
    mesh=_mesh,
    scratch_types=[
        pltpu.VMEM((IPG,), jnp.int32),
        pltpu.VMEM((RPG, E), jnp.float32),
        pltpu.VMEM((RPG, E), jnp.float32),
        pltpu.SemaphoreType.DMA,
    ],
    compiler_params=pltpu.CompilerParams(use_tc_tiling_on_sc=False),
)
def _sc_pool(xg, table, out, idx_v, gbuf, acc, sem):
    wid = lax.axis_index("s") * NC + lax.axis_index("c")
    for g in range(GPW):
        gg = wid * GPW + g
        pltpu.sync_copy(xg.at[pl.ds(gg * IPG, IPG)], idx_v)

        pltpu.async_copy(table.at[idx_v.at[pl.ds(0, RPG)]], gbuf, sem).wait()

        def init_body(r, carry):
            acc[r, pl.ds(0, 16)] = gbuf[r, pl.ds(0, 16)]
            acc[r, pl.ds(16, 16)] = gbuf[r, pl.ds(16, 16)]
            return carry

        lax.fori_loop(0, RPG, init_body, 0, unroll=4)

        def h_body(h, carry):
            pltpu.async_copy(
                table.at[idx_v.at[pl.ds(h * RPG, RPG)]], gbuf, sem
            ).wait()

            def r_body(r, c2):
                plsc.addupdate(acc.at[r, pl.ds(0, 16)], gbuf[r, pl.ds(0, 16)])
                plsc.addupdate(acc.at[r, pl.ds(16, 16)], gbuf[r, pl.ds(16, 16)])
                return c2

            lax.fori_loop(0, RPG, r_body, 0, unroll=4)
            return carry

        lax.fori_loop(1, H, h_body, 0)
        pltpu.sync_copy(acc, out.at[pl.ds(gg * RPG, RPG)])


def _mlp_body(p_ref, w1_ref, b1_ref, w2_ref, b2_ref, o_ref):
    h = jnp.dot(p_ref[...], w1_ref[...], preferred_element_type=jnp.float32)
    h = h + b1_ref[...]
    o = jnp.dot(h, w2_ref[...], preferred_element_type=jnp.float32)
    o_ref[...] = o + b2_ref[...]


_MLP_BLOCK = 2048
_mlp = pl.pallas_call(
    _mlp_body,
    grid=(B // _MLP_BLOCK,),
    in_specs=[
        pl.BlockSpec((_MLP_BLOCK, E), lambda i: (i, 0)),
        pl.BlockSpec((E, HID), lambda i: (0, 0)),
        pl.BlockSpec((1, HID), lambda i: (0, 0)),
        pl.BlockSpec((HID, OUT), lambda i: (0, 0)),
        pl.BlockSpec((1, OUT), lambda i: (0, 0)),
    ],
    out_specs=pl.BlockSpec((_MLP_BLOCK, OUT), lambda i: (i, 0)),
    out_shape=jax.ShapeDtypeStruct((B, OUT), jnp.float32),
)


@jax.jit
def kernel(x, embeddings, W1, b1, W2, b2):
    xg = (
        x.astype(jnp.int32)
        .reshape(G, RPG, H)
        .transpose(0, 2, 1)
        .reshape(-1)
    )
    pooled = _sc_pool(xg, embeddings)
    return _mlp(pooled, W1, b1.reshape(1, HID), W2, b2.reshape(1, OUT))


# speedup vs baseline: 17.1053x; 2.0515x over previous
import functools

import jax
import jax.numpy as jnp
from jax import lax
from jax.experimental import pallas as pl
from jax.experimental.pallas import tpu as pltpu
from jax.experimental.pallas import tpu_sc as plsc

B = 16384
H = 200
E = 32
HID = 128
OUT = 2

NC, NS = 2, 16
NW = NC * NS
RPG = 128
G = B // RPG
GPW = G // NW
IPG = RPG * H

_mesh = plsc.VectorSubcoreMesh(
    core_axis_name="c", subcore_axis_name="s", num_cores=NC, num_subcores=NS
)


@functools.partial(
    pl.kernel,
    out_type=jax.ShapeDtypeStruct((B, E), jnp.float32),
    mesh=_mesh,
    scratch_types=[
        pltpu.VMEM((IPG,), jnp.int32),
        pltpu.VMEM((RPG, E), jnp.float32),
        pltpu.SemaphoreType.DMA,
    ],
    compiler_params=pltpu.CompilerParams(use_tc_tiling_on_sc=False),
)
def _sc_pool(xg, table, out, idx_v, acc, sem):
    wid = lax.axis_index("s") * NC + lax.axis_index("c")
    zero = jnp.zeros((16,), jnp.float32)
    for g in range(GPW):
        gg = wid * GPW + g
        pltpu.sync_copy(xg.at[pl.ds(gg * IPG, IPG)], idx_v)

        def zero_body(r, carry):
            acc[r, pl.ds(0, 16)] = zero
            acc[r, pl.ds(16, 16)] = zero
            return carry

        lax.fori_loop(0, RPG, zero_body, 0, unroll=8)

        def fire_body(h, carry):
            pltpu.async_copy(
                table.at[idx_v.at[pl.ds(h * RPG, RPG)]], acc, sem, add=True
            )
            return carry

        lax.fori_loop(0, H, fire_body, 0)

        def drain_body(h, carry):
            pltpu.make_async_copy(table.at[idx_v.at[pl.ds(0, RPG)]], acc, sem).wait()
            return carry

        lax.fori_loop(0, H, drain_body, 0)

        pltpu.sync_copy(acc, out.at[pl.ds(gg * RPG, RPG)])


def _mlp_body(p_ref, w1_ref, b1_ref, w2_ref, b2_ref, o_ref):
    h = jnp.dot(p_ref[...], w1_ref[...], preferred_element_type=jnp.float32)
    h = h + b1_ref[...]
    o = jnp.dot(h, w2_ref[...], preferred_element_type=jnp.float32)
    o_ref[...] = o + b2_ref[...]


_MLP_BLOCK = 2048
_mlp = pl.pallas_call(
    _mlp_body,
    grid=(B // _MLP_BLOCK,),
    in_specs=[
        pl.BlockSpec((_MLP_BLOCK, E), lambda i: (i, 0)),
        pl.BlockSpec((E, HID), lambda i: (0, 0)),
        pl.BlockSpec((1, HID), lambda i: (0, 0)),
        pl.BlockSpec((HID, OUT), lambda i: (0, 0)),
        pl.BlockSpec((1, OUT), lambda i: (0, 0)),
    ],
    out_specs=pl.BlockSpec((_MLP_BLOCK, OUT), lambda i: (i, 0)),
    out_shape=jax.ShapeDtypeStruct((B, OUT), jnp.float32),
)


@jax.jit
def kernel(x, embeddings, W1, b1, W2, b2):
    xg = (
        x.astype(jnp.int32)
        .reshape(G, RPG, H)
        .transpose(0, 2, 1)
        .reshape(-1)
    )
    pooled = _sc_pool(xg, embeddings)
    return _mlp(pooled, W1, b1.reshape(1, HID), W2, b2.reshape(1, OUT))
